# baseline (speedup 1.0000x reference)
import jax
import jax.numpy as jnp
from jax.experimental import pallas as pl


def _gcn_fused(a_ref, f_ref, w1_ref, b1_ref, w2_ref, b2_ref, out_ref):
    a = a_ref[0]
    n = a.shape[0]
    rows = jax.lax.broadcasted_iota(jnp.int32, (n, n), 0)
    cols = jax.lax.broadcasted_iota(jnp.int32, (n, n), 1)
    eye = jnp.where(rows == cols, jnp.float32(1.0), jnp.float32(0.0))
    adj = jnp.abs(a) + eye
    deg = jnp.sum(adj, axis=1, keepdims=True)
    d = jax.lax.rsqrt(deg)

    f = f_ref[0]
    s1 = jnp.dot(f, w1_ref[...], preferred_element_type=jnp.float32)
    p1 = d * jnp.dot(adj, d * s1, preferred_element_type=jnp.float32)
    h1 = jnp.maximum(p1 + b1_ref[...], 0.0)

    s2 = jnp.dot(h1, w2_ref[...], preferred_element_type=jnp.float32)
    p2 = d * jnp.dot(adj, d * s2, preferred_element_type=jnp.float32)
    out_ref[0] = jnp.maximum(p2 + b2_ref[...], 0.0)


def kernel(a, f, W1, b1, W2, b2):
    B, N, _ = a.shape
    C = f.shape[2]
    H = W2.shape[1]
    return pl.pallas_call(
        _gcn_fused,
        grid=(B,),
        in_specs=[
            pl.BlockSpec((1, N, N), lambda b: (b, 0, 0)),
            pl.BlockSpec((1, N, C), lambda b: (b, 0, 0)),
            pl.BlockSpec((C, H), lambda b: (0, 0)),
            pl.BlockSpec((1, H), lambda b: (0, 0)),
            pl.BlockSpec((H, H), lambda b: (0, 0)),
            pl.BlockSpec((1, H), lambda b: (0, 0)),
        ],
        out_specs=pl.BlockSpec((1, N, H), lambda b: (b, 0, 0)),
        out_shape=jax.ShapeDtypeStruct((B, N, H), jnp.float32),
    )(a, f, W1, b1.reshape(1, -1), W2, b2.reshape(1, -1))


# speedup vs baseline: 1.3573x; 1.3573x over previous
import functools

import jax
import jax.numpy as jnp
from jax.experimental import pallas as pl

_G = 8


def _gcn_fused(a_ref, f_ref, w1_ref, b1_ref, w2_ref, b2_ref, out_ref, *, G):
    n = a_ref.shape[1]
    rows = jax.lax.broadcasted_iota(jnp.int32, (n, n), 0)
    cols = jax.lax.broadcasted_iota(jnp.int32, (n, n), 1)
    eye = jnp.where(rows == cols, jnp.float32(1.0), jnp.float32(0.0))
    adj = jnp.abs(a_ref[...]) + eye[None]
    d = jax.lax.rsqrt(jnp.sum(adj, axis=2, keepdims=True))

    w1 = w1_ref[...]
    w2 = w2_ref[...]
    b1 = b1_ref[...]
    b2 = b2_ref[...]
    for g in range(G):
        dg = d[g]
        s1 = jnp.dot(f_ref[g], w1, preferred_element_type=jnp.float32)
        p1 = dg * jnp.dot(adj[g], dg * s1, preferred_element_type=jnp.float32)
        h1 = jnp.maximum(p1 + b1, 0.0)
        s2 = jnp.dot(h1, w2, preferred_element_type=jnp.float32)
        p2 = dg * jnp.dot(adj[g], dg * s2, preferred_element_type=jnp.float32)
        out_ref[g] = jnp.maximum(p2 + b2, 0.0)


def kernel(a, f, W1, b1, W2, b2):
    B, N, _ = a.shape
    C = f.shape[2]
    H = W2.shape[1]
    G = _G
    return pl.pallas_call(
        functools.partial(_gcn_fused, G=G),
        grid=(B // G,),
        in_specs=[
            pl.BlockSpec((G, N, N), lambda b: (b, 0, 0)),
            pl.BlockSpec((G, N, C), lambda b: (b, 0, 0)),
            pl.BlockSpec((C, H), lambda b: (0, 0)),
            pl.BlockSpec((1, H), lambda b: (0, 0)),
            pl.BlockSpec((H, H), lambda b: (0, 0)),
            pl.BlockSpec((1, H), lambda b: (0, 0)),
        ],
        out_specs=pl.BlockSpec((G, N, H), lambda b: (b, 0, 0)),
        out_shape=jax.ShapeDtypeStruct((B, N, H), jnp.float32),
    )(a, f, W1, b1.reshape(1, -1), W2, b2.reshape(1, -1))


# speedup vs baseline: 1.3590x; 1.0013x over previous
import functools

import jax
import jax.numpy as jnp
from jax.experimental import pallas as pl

_G = 16


def _gcn_fused(a_ref, f_ref, w1_ref, b1_ref, w2_ref, b2_ref, out_ref, *, G):
    n = a_ref.shape[1]
    rows = jax.lax.broadcasted_iota(jnp.int32, (n, n), 0)
    cols = jax.lax.broadcasted_iota(jnp.int32, (n, n), 1)
    eye = jnp.where(rows == cols, jnp.float32(1.0), jnp.float32(0.0))
    adj = jnp.abs(a_ref[...]) + eye[None]
    d = jax.lax.rsqrt(jnp.sum(adj, axis=2, keepdims=True))

    w1 = w1_ref[...]
    w2 = w2_ref[...]
    b1 = b1_ref[...]
    b2 = b2_ref[...]
    for g in range(G):
        dg = d[g]
        s1 = jnp.dot(f_ref[g], w1, preferred_element_type=jnp.float32)
        p1 = dg * jnp.dot(adj[g], dg * s1, preferred_element_type=jnp.float32)
        h1 = jnp.maximum(p1 + b1, 0.0)
        s2 = jnp.dot(h1, w2, preferred_element_type=jnp.float32)
        p2 = dg * jnp.dot(adj[g], dg * s2, preferred_element_type=jnp.float32)
        out_ref[g] = jnp.maximum(p2 + b2, 0.0)


def kernel(a, f, W1, b1, W2, b2):
    B, N, _ = a.shape
    C = f.shape[2]
    H = W2.shape[1]
    G = _G
    return pl.pallas_call(
        functools.partial(_gcn_fused, G=G),
        grid=(B // G,),
        in_specs=[
            pl.BlockSpec((G, N, N), lambda b: (b, 0, 0)),
            pl.BlockSpec((G, N, C), lambda b: (b, 0, 0)),
            pl.BlockSpec((C, H), lambda b: (0, 0)),
            pl.BlockSpec((1, H), lambda b: (0, 0)),
            pl.BlockSpec((H, H), lambda b: (0, 0)),
            pl.BlockSpec((1, H), lambda b: (0, 0)),
        ],
        out_specs=pl.BlockSpec((G, N, H), lambda b: (b, 0, 0)),
        out_shape=jax.ShapeDtypeStruct((B, N, H), jnp.float32),
    )(a, f, W1, b1.reshape(1, -1), W2, b2.reshape(1, -1))


# speedup vs baseline: 2.1341x; 1.5703x over previous
import functools

import jax
import jax.numpy as jnp
from jax.experimental import pallas as pl

_G = 16


def _gcn_fused(a_ref, f_ref, w1_ref, b1_ref, w2_ref, b2_ref, out_ref, *, G):
    n = a_ref.shape[1]
    rows = jax.lax.broadcasted_iota(jnp.int32, (n, n), 0)
    cols = jax.lax.broadcasted_iota(jnp.int32, (n, n), 1)
    eye = jnp.where(rows == cols, jnp.float32(1.0), jnp.float32(0.0))
    adj = jnp.abs(a_ref[...]) + eye[None]
    d = jax.lax.rsqrt(jnp.sum(adj, axis=2, keepdims=True))

    w1 = w1_ref[...]
    w2 = w2_ref[...]
    b1 = b1_ref[...]
    b2 = b2_ref[...]
    s1 = [jnp.dot(f_ref[g], w1, preferred_element_type=jnp.float32)
          for g in range(G)]
    t1 = [d[g] * s1[g] for g in range(G)]
    p1 = [jnp.dot(adj[g], t1[g], preferred_element_type=jnp.float32)
          for g in range(G)]
    h1 = [jnp.maximum(d[g] * p1[g] + b1, 0.0) for g in range(G)]
    s2 = [jnp.dot(h1[g], w2, preferred_element_type=jnp.float32)
          for g in range(G)]
    t2 = [d[g] * s2[g] for g in range(G)]
    p2 = [jnp.dot(adj[g], t2[g], preferred_element_type=jnp.float32)
          for g in range(G)]
    for g in range(G):
        out_ref[g] = jnp.maximum(d[g] * p2[g] + b2, 0.0)


def kernel(a, f, W1, b1, W2, b2):
    B, N, _ = a.shape
    C = f.shape[2]
    H = W2.shape[1]
    G = _G
    return pl.pallas_call(
        functools.partial(_gcn_fused, G=G),
        grid=(B // G,),
        in_specs=[
            pl.BlockSpec((G, N, N), lambda b: (b, 0, 0)),
            pl.BlockSpec((G, N, C), lambda b: (b, 0, 0)),
            pl.BlockSpec((C, H), lambda b: (0, 0)),
            pl.BlockSpec((1, H), lambda b: (0, 0)),
            pl.BlockSpec((H, H), lambda b: (0, 0)),
            pl.BlockSpec((1, H), lambda b: (0, 0)),
        ],
        out_specs=pl.BlockSpec((G, N, H), lambda b: (b, 0, 0)),
        out_shape=jax.ShapeDtypeStruct((B, N, H), jnp.float32),
    )(a, f, W1, b1.reshape(1, -1), W2, b2.reshape(1, -1))
